# Initial kernel scaffold; baseline (speedup 1.0000x reference)
import jax
import jax.numpy as jnp
from jax.experimental import pallas as pl


def kernel(X, tf_prob_logits, tf_prob_sample, is_fit, X_type):
    raise NotImplementedError("write your pallas kernel here")


# speedup vs baseline: 5.9614x; 5.9614x over previous
import functools

import jax
import jax.numpy as jnp
from jax.experimental import pallas as pl
from jax.experimental.pallas import tpu as pltpu

_B, _F = 16384, 128
_BLK = 2048


def _body(w_ref, x_ref, o_ref):
    x = x_ref[...]
    w0 = w_ref[0:1, :]
    w1 = w_ref[1:2, :]
    w2 = w_ref[2:3, :]
    w3 = w_ref[3:4, :]
    sgn = jnp.sign(x)
    ax = jnp.abs(x)
    out = w0 * x + w3 * (x * x)
    out = out + w1 * (sgn * jnp.log1p(ax))
    out = out + w2 * (sgn * jnp.sqrt(ax))
    o_ref[...] = out


@functools.partial(jax.jit, static_argnames=("blk",))
def _fused(X, wT, blk):
    grid = (X.shape[0] // blk,)
    return pl.pallas_call(
        _body,
        grid=grid,
        in_specs=[
            pl.BlockSpec((8, _F), lambda i: (0, 0)),
            pl.BlockSpec((blk, _F), lambda i: (i, 0)),
        ],
        out_specs=pl.BlockSpec((blk, _F), lambda i: (i, 0)),
        out_shape=jax.ShapeDtypeStruct(X.shape, X.dtype),
    )(wT, X)


def kernel(X, tf_prob_logits, tf_prob_sample, is_fit, X_type):
    wT = jnp.zeros((8, _F), jnp.float32).at[0:4, :].set(tf_prob_sample.T)
    return _fused(X, wT, _BLK)


# speedup vs baseline: 6.8747x; 1.1532x over previous
import functools

import jax
import jax.numpy as jnp
from jax import lax
from jax.experimental import pallas as pl

_B, _F = 16384, 128
_BLK = 2048
_LN2 = 0.6931471805599453


def _body(w_ref, x_ref, o_ref):
    x = x_ref[...]
    m1 = w_ref[1:2, :] > 0.5
    m2 = w_ref[2:3, :] > 0.5
    m3 = w_ref[3:4, :] > 0.5
    xb = lax.bitcast_convert_type(x, jnp.int32)
    sbit = jnp.bitwise_and(xb, jnp.int32(-2147483648))
    ab = jnp.bitwise_and(xb, jnp.int32(0x7FFFFFFF))
    ax = lax.bitcast_convert_type(ab, jnp.float32)
    l = lax.log(ax + 1.0)
    t1 = lax.bitcast_convert_type(
        jnp.bitwise_or(lax.bitcast_convert_type(l, jnp.int32), sbit), jnp.float32
    )
    s = ax * lax.rsqrt(ax + 1e-35)
    t2 = lax.bitcast_convert_type(
        jnp.bitwise_or(lax.bitcast_convert_type(s, jnp.int32), sbit), jnp.float32
    )
    out = jnp.where(m1, t1, x)
    out = jnp.where(m2, t2, out)
    out = jnp.where(m3, x * x, out)
    o_ref[...] = out


@functools.partial(jax.jit, static_argnames=("blk",))
def _fused(X, wT, blk):
    grid = (X.shape[0] // blk,)
    return pl.pallas_call(
        _body,
        grid=grid,
        in_specs=[
            pl.BlockSpec((8, _F), lambda i: (0, 0)),
            pl.BlockSpec((blk, _F), lambda i: (i, 0)),
        ],
        out_specs=pl.BlockSpec((blk, _F), lambda i: (i, 0)),
        out_shape=jax.ShapeDtypeStruct(X.shape, X.dtype),
    )(wT, X)


def kernel(X, tf_prob_logits, tf_prob_sample, is_fit, X_type):
    wT = jnp.zeros((8, _F), jnp.float32).at[0:4, :].set(tf_prob_sample.T)
    return _fused(X, wT, _BLK)
